# baseline (speedup 1.0000x reference)
import functools

import jax
import jax.numpy as jnp
from jax.experimental import pallas as pl
from jax.experimental.pallas import tpu as pltpu

KNN = 12
NOFF = 27

_OFFS = [(float(u), float(v), float(w))
         for u in (-1.0, 0.0, 1.0)
         for v in (-1.0, 0.0, 1.0)
         for w in (-1.0, 0.0, 1.0)]


def _rbf(v):
    u = jax.lax.bitcast_convert_type(v, jnp.uint32)
    lsb = (u >> 16) & jnp.uint32(1)
    u = (u + jnp.uint32(0x7FFF) + lsb) & jnp.uint32(0xFFFF0000)
    return jax.lax.bitcast_convert_type(u, jnp.float32)


def _body(cell_ref, xr_ref, xc_ref, xtr_ref, xtc_ref,
          diff_ref, psum_ref, d2_scr, *, a):
    cb = _rbf(cell_ref[...].reshape(3, 3))
    CB = [[cb[i:i + 1, j:j + 1] for j in range(3)] for i in range(3)]
    xR = xr_ref[...].reshape(3, a)
    xtR = xtr_ref[...].reshape(3, a)
    xC = xc_ref[...].reshape(a, 3)
    xtC = xtc_ref[...].reshape(a, 3)
    xrow = [xR[c:c + 1, :] for c in range(3)]
    xtrow = [xtR[c:c + 1, :] for c in range(3)]
    xcol = [xC[:, c:c + 1] for c in range(3)]
    xtcol = [xtC[:, c:c + 1] for c in range(3)]

    ub = [_rbf(xcol[c]) for c in range(3)]
    xi = [(ub[0] * CB[0][d] + ub[1] * CB[1][d]) + ub[2] * CB[2][d]
          for d in range(3)]
    ni = (xi[0] * xi[0] + xi[1] * xi[1]) + xi[2] * xi[2]
    xib = [_rbf(xi[d]) for d in range(3)]

    ii = jax.lax.broadcasted_iota(jnp.int32, (a, a), 0)
    jj = jax.lax.broadcasted_iota(jnp.int32, (a, a), 1)
    eye12 = jnp.where(ii == jj, jnp.float32(1e12), jnp.float32(0.0))

    for o in range(NOFF):
        off = _OFFS[o]
        uj = [_rbf(xrow[c] + off[c]) for c in range(3)]
        xj = [(uj[0] * CB[0][d] + uj[1] * CB[1][d]) + uj[2] * CB[2][d]
              for d in range(3)]
        nj = (xj[0] * xj[0] + xj[1] * xj[1]) + xj[2] * xj[2]
        xjb = [_rbf(xj[d]) for d in range(3)]
        dot = (xib[0] * xjb[0] + xib[1] * xjb[1]) + xib[2] * xjb[2]
        d2s = (ni + nj) - 2.0 * dot
        if o == 13:
            d2s = d2s + eye12
        d2_scr[:, o * a:(o + 1) * a] = d2s

    lane = jax.lax.broadcasted_iota(jnp.int32, (1, a * NOFF), 1)
    jrow = ((lane % a) * NOFF + lane // a).astype(jnp.float32)

    big = jnp.float32(3.0e38)
    sels = []
    d2v = d2_scr[...]
    for _ in range(KNN):
        m = jnp.min(d2v, axis=1, keepdims=True)
        sel = jnp.min(jnp.where(d2v == m, jrow, big), axis=1, keepdims=True)
        sels.append(sel)
        d2v = jnp.where(jrow == sel, big, d2v)

    lane_f = jax.lax.broadcasted_iota(jnp.int32, (1, a), 1).astype(jnp.float32)
    cols = []
    for k in range(KNN):
        sel = sels[k]
        j = jnp.floor(sel / 27.0)
        o = sel - 27.0 * j
        q0 = jnp.floor(o / 9.0)
        r0 = o - 9.0 * q0
        q1 = jnp.floor(r0 / 3.0)
        q2 = r0 - 3.0 * q1
        offk = [q0 - 1.0, q1 - 1.0, q2 - 1.0]
        hit = lane_f == j
        for c in range(3):
            gx = jnp.sum(jnp.where(hit, xrow[c], 0.0), axis=1, keepdims=True)
            gxt = jnp.sum(jnp.where(hit, xtrow[c], 0.0), axis=1, keepdims=True)
            e = (gx + offk[c]) - xcol[c]
            et = (gxt + offk[c]) - xtcol[c]
            cols.append(jnp.abs(et - e))
    diffb = jnp.concatenate(cols, axis=1)
    diff_ref[...] = diffb.reshape(1, a, 3 * KNN)

    psum_ref[...] = jnp.sum(jnp.sum(diffb, axis=1, keepdims=True),
                            axis=0, keepdims=True).reshape(1, 1, 1)


def kernel(cell, x, x_tilde, num_atoms):
    b = cell.shape[0]
    n = x.shape[0]
    a = n // b
    xb = x.reshape(b, a, 3)
    xtb = x_tilde.reshape(b, a, 3)
    xr = jnp.swapaxes(xb, 1, 2)
    xtr = jnp.swapaxes(xtb, 1, 2)
    body = functools.partial(_body, a=a)
    diff4, psums = pl.pallas_call(
        body,
        grid=(b,),
        in_specs=[
            pl.BlockSpec((1, 3, 3), lambda i: (i, 0, 0)),
            pl.BlockSpec((1, 3, a), lambda i: (i, 0, 0)),
            pl.BlockSpec((1, a, 3), lambda i: (i, 0, 0)),
            pl.BlockSpec((1, 3, a), lambda i: (i, 0, 0)),
            pl.BlockSpec((1, a, 3), lambda i: (i, 0, 0)),
        ],
        out_specs=(
            pl.BlockSpec((1, a, 3 * KNN), lambda i: (i, 0, 0)),
            pl.BlockSpec((1, 1, 1), lambda i: (i, 0, 0)),
        ),
        scratch_shapes=[
            pltpu.VMEM((a, a * NOFF), jnp.float32),
        ],
        out_shape=(
            jax.ShapeDtypeStruct((b, a, 3 * KNN), jnp.float32),
            jax.ShapeDtypeStruct((b, 1, 1), jnp.float32),
        ),
        compiler_params=pltpu.CompilerParams(
            dimension_semantics=("parallel",)),
    )(cell, xr, xb, xtr, xtb)
    loss = jnp.sum(psums) / jnp.float32(b * a * KNN * 3)
    return (loss, diff4.reshape(n * KNN, 3))
